# baseline (speedup 1.0000x reference)
import functools

import jax
import jax.numpy as jnp
from jax import lax
from jax.experimental import pallas as pl
from jax.experimental.pallas import tpu as pltpu
from jax.experimental.pallas import tpu_sc as plsc

_B = 16384
_T = 50
_D = 32
_CH = 256
_V = 1000000


def _make_lookup():
    info = plsc.get_sparse_core_info()
    nc, ns = info.num_cores, info.num_subcores
    nw = nc * ns
    b_per_w = _B // nw
    n_ids = _T * b_per_w
    n_chunks = n_ids // _CH
    n_quads = n_chunks // 4
    mesh = plsc.VectorSubcoreMesh(core_axis_name="c", subcore_axis_name="s")

    @functools.partial(
        pl.kernel,
        out_type=jax.ShapeDtypeStruct((_T * _D, _B), jnp.float32),
        mesh=mesh,
        compiler_params=pltpu.CompilerParams(
            use_tc_tiling_on_sc=False, needs_layout_passes=False
        ),
        scratch_types=(
            [pltpu.VMEM((n_ids,), jnp.int32)]
            + [pltpu.VMEM((2 * _CH,), jnp.int32) for _ in range(4)]
            + [pltpu.VMEM((2 * _CH, 16), jnp.float32) for _ in range(4)]
            + [pltpu.VMEM((_D, _CH), jnp.float32) for _ in range(4)]
            + [pltpu.SemaphoreType.DMA]
            + [pltpu.SemaphoreType.DMA for _ in range(4)]
            + [pltpu.SemaphoreType.DMA for _ in range(4)]
        ),
    )
    def lookup(ids_hbm, packs_hbm, out_hbm, ids_v,
               ipa, ipb, ipc, ipd, bufa, bufb, bufc, bufd,
               oba, obb, obc, obd, sem_i,
               sga, sgb, sgc, sgd, ssa, ssb, ssc, ssd):
        ips = [ipa, ipb, ipc, ipd]
        bufs = [bufa, bufb, bufc, bufd]
        obs = [oba, obb, obc, obd]
        sgs = [sga, sgb, sgc, sgd]
        sss = [ssa, ssb, ssc, ssd]
        wid = lax.axis_index("s") * nc + lax.axis_index("c")
        col0 = wid * b_per_w

        def stage(t, carry):
            pltpu.async_copy(
                ids_hbm.at[pl.ds(t * _B + col0, b_per_w)],
                ids_v.at[pl.ds(t * b_per_w, b_per_w)],
                sem_i,
            )
            return carry

        lax.fori_loop(0, _T, stage, 0)
        pltpu.make_async_copy(ids_hbm.at[pl.ds(0, n_ids)], ids_v, sem_i).wait()

        iota16 = lax.iota(jnp.int32, 16)
        cols = [iota16 * 0 + dm for dm in range(16)]
        rows_e = [2 * (iota16 + g * 16) for g in range(_CH // 16)]
        scat_e = [2 * iota16 + 32 * g for g in range(_CH // 16)]

        def build_ip(c, ip):
            for g in range(_CH // 16):
                v = ids_v[pl.ds(c * _CH + g * 16, 16)]
                v2 = v + v
                plsc.store_scatter(ip, [scat_e[g]], v2)
                plsc.store_scatter(ip, [scat_e[g] + 1], v2 + 1)

        def gather(ip, buf, sem):
            pltpu.async_copy(packs_hbm.at[ip], buf, sem)

        def wait_g(ip, buf, sem):
            pltpu.make_async_copy(packs_hbm.at[ip], buf, sem).wait()

        def transpose(buf, ob):
            for g in range(_CH // 16):
                re = rows_e[g]
                ro = re + 1
                for d in range(_D):
                    rows = re if d < 16 else ro
                    x = plsc.load_gather(buf, [rows, cols[d % 16]])
                    ob[d, pl.ds(g * 16, 16)] = x

        def out_slice(c):
            t = c // 2
            b0 = col0 + (c % 2) * _CH
            return out_hbm.at[pl.ds(_D * t, _D), pl.ds(b0, _CH)]

        def store(c, ob, sem):
            pltpu.async_copy(ob, out_slice(c), sem)

        def wait_s(c, ob, sem):
            pltpu.make_async_copy(ob, out_slice(c), sem).wait()

        for k in range(4):
            build_ip(k, ips[k])
            gather(ips[k], bufs[k], sgs[k])

        def body(j, carry):
            c_base = 4 * j
            for k in range(4):
                c = c_base + k
                wait_g(ips[k], bufs[k], sgs[k])

                @pl.when(j > 0)
                def _():
                    wait_s(c - 4, obs[k], sss[k])

                transpose(bufs[k], obs[k])
                store(c, obs[k], sss[k])

                @pl.when(j < n_quads - 1)
                def _():
                    build_ip(c + 4, ips[k])
                    gather(ips[k], bufs[k], sgs[k])

            return carry

        lax.fori_loop(0, n_quads, body, 0)
        for k in range(4):
            wait_s(n_chunks - 4 + k, obs[k], sss[k])

    return lookup


_lookup = _make_lookup()


@jax.jit
def kernel(token_ids, emb):
    ids_t_major = token_ids.T.reshape(-1).astype(jnp.int32)
    packs = emb.reshape(2 * _V, 16)
    out2 = _lookup(ids_t_major, packs)
    return jnp.transpose(out2.reshape(_T, _D, _B), (2, 0, 1))


# speedup vs baseline: 1.9315x; 1.9315x over previous
import functools

import jax
import jax.numpy as jnp
from jax import lax
from jax.experimental import pallas as pl
from jax.experimental.pallas import tpu as pltpu
from jax.experimental.pallas import tpu_sc as plsc

_B = 16384
_T = 50
_D = 32
_CH = 256
_V = 1000000


def _make_lookup():
    info = plsc.get_sparse_core_info()
    nc, ns = info.num_cores, info.num_subcores
    nw = nc * ns
    b_per_w = _B // nw
    n_ids = _T * b_per_w
    n_chunks = n_ids // _CH
    n_quads = n_chunks // 4
    mesh = plsc.VectorSubcoreMesh(core_axis_name="c", subcore_axis_name="s")

    @functools.partial(
        pl.kernel,
        out_type=jax.ShapeDtypeStruct((_T * _D, _B), jnp.float32),
        mesh=mesh,
        compiler_params=pltpu.CompilerParams(
            use_tc_tiling_on_sc=False,
            needs_layout_passes=False,
            disable_bounds_checks=True,
        ),
        scratch_types=(
            [pltpu.VMEM((n_ids,), jnp.int32)]
            + [pltpu.VMEM((2 * _CH,), jnp.int32) for _ in range(4)]
            + [pltpu.VMEM((2 * _CH, 16), jnp.float32) for _ in range(4)]
            + [pltpu.VMEM((_D, _CH), jnp.float32) for _ in range(4)]
            + [pltpu.SemaphoreType.DMA]
            + [pltpu.SemaphoreType.DMA for _ in range(4)]
            + [pltpu.SemaphoreType.DMA for _ in range(4)]
        ),
    )
    def lookup(ids_hbm, packs_hbm, out_hbm, ids_v,
               ipa, ipb, ipc, ipd, bufa, bufb, bufc, bufd,
               oba, obb, obc, obd, sem_i,
               sga, sgb, sgc, sgd, ssa, ssb, ssc, ssd):
        ips = [ipa, ipb, ipc, ipd]
        bufs = [bufa, bufb, bufc, bufd]
        obs = [oba, obb, obc, obd]
        sgs = [sga, sgb, sgc, sgd]
        sss = [ssa, ssb, ssc, ssd]
        wid = lax.axis_index("s") * nc + lax.axis_index("c")
        col0 = wid * b_per_w

        def stage(t, carry):
            pltpu.async_copy(
                ids_hbm.at[pl.ds(t * _B + col0, b_per_w)],
                ids_v.at[pl.ds(t * b_per_w, b_per_w)],
                sem_i,
            )
            return carry

        lax.fori_loop(0, _T, stage, 0)
        pltpu.make_async_copy(ids_hbm.at[pl.ds(0, n_ids)], ids_v, sem_i).wait()

        iota16 = lax.iota(jnp.int32, 16)
        cols = [iota16 * 0 + dm for dm in range(16)]
        rows_e = [2 * (iota16 + g * 16) for g in range(_CH // 16)]
        scat_e = [2 * iota16 + 32 * g for g in range(_CH // 16)]

        def build_ip(c, ip):
            for g in range(_CH // 16):
                v = ids_v[pl.ds(c * _CH + g * 16, 16)]
                v2 = v + v
                plsc.store_scatter(ip, [scat_e[g]], v2)
                plsc.store_scatter(ip, [scat_e[g] + 1], v2 + 1)

        def gather(ip, buf, sem):
            pltpu.async_copy(packs_hbm.at[ip], buf, sem)

        def wait_g(ip, buf, sem):
            pltpu.make_async_copy(packs_hbm.at[ip], buf, sem).wait()

        def transpose(buf, ob):
            @functools.partial(plsc.parallel_loop, 0, _CH // 16, unroll=2)
            def _(g):
                re = 2 * (iota16 + g * 16)
                ro = re + 1
                for d in range(_D):
                    rows = re if d < 16 else ro
                    x = plsc.load_gather(buf, [rows, cols[d % 16]])
                    ob[d, pl.ds(g * 16, 16)] = x

        def out_slice(c):
            t = c // 2
            b0 = col0 + (c % 2) * _CH
            return out_hbm.at[pl.ds(_D * t, _D), pl.ds(b0, _CH)]

        def store(c, ob, sem):
            pltpu.async_copy(ob, out_slice(c), sem)

        def wait_s(c, ob, sem):
            pltpu.make_async_copy(ob, out_slice(c), sem).wait()

        for k in range(4):
            build_ip(k, ips[k])
            gather(ips[k], bufs[k], sgs[k])

        def body(j, carry):
            c_base = 4 * j
            for k in range(4):
                c = c_base + k
                wait_g(ips[k], bufs[k], sgs[k])

                @pl.when(j > 0)
                def _():
                    wait_s(c - 4, obs[k], sss[k])

                transpose(bufs[k], obs[k])
                store(c, obs[k], sss[k])

                @pl.when(j < n_quads - 1)
                def _():
                    build_ip(c + 4, ips[k])
                    gather(ips[k], bufs[k], sgs[k])

            return carry

        lax.fori_loop(0, n_quads, body, 0)
        for k in range(4):
            wait_s(n_chunks - 4 + k, obs[k], sss[k])

    return lookup


_lookup = _make_lookup()


@jax.jit
def kernel(token_ids, emb):
    ids_t_major = token_ids.T.reshape(-1).astype(jnp.int32)
    packs = emb.reshape(2 * _V, 16)
    out2 = _lookup(ids_t_major, packs)
    return jnp.transpose(out2.reshape(_T, _D, _B), (2, 0, 1))
